# Initial kernel scaffold; baseline (speedup 1.0000x reference)
import jax
import jax.numpy as jnp
from jax.experimental import pallas as pl


def kernel(x, edge_index, W1, b1, W2, b2, W3, b3, W4, b4):
    raise NotImplementedError("write your pallas kernel here")


# speedup vs baseline: 33.9603x; 33.9603x over previous
import functools

import jax
import jax.numpy as jnp
from jax import lax
from jax.experimental import pallas as pl
from jax.experimental.pallas import tpu as pltpu
from jax.experimental.pallas import tpu_sc as plsc

N = 100000
E = 1600000
NPAD = 100352
ROWS = E // 128
SUB = 128
JW = 8
NC = 2
NS = 16
RPT = NPAD // NS
ZCH = 784
CPT = NPAD // 32
BLK = 2048
GRID = NPAD // BLK


@functools.cache
def _mesh():
    return plsc.VectorSubcoreMesh(core_axis_name="c", subcore_axis_name="s",
                                  num_cores=NC, num_subcores=NS)


def _fill_rows(ref, nrows, val):
    def body(i, carry):
        ref[i, :] = jnp.full((16,), val, jnp.float32)
        return carry
    lax.fori_loop(0, nrows, body, 0)


def _fill_flat(ref, n, val):
    def body(i, carry):
        ref[pl.ds(i * 16, 16)] = jnp.full((16,), val, jnp.float32)
        return carry
    lax.fori_loop(0, n // 16, body, 0)


def _zero_acc_rows(rows, acc, s):
    _fill_rows(rows, ZCH, 0.0)
    for k in range(RPT // ZCH):
        pltpu.sync_copy(rows.at[pl.ds(0, ZCH)],
                        acc.at[pl.ds(s * RPT + k * ZCH, ZCH)])


def _deg_body(ei_hbm, out_hbm, idxd, ones_v, zbuf, acc, sem):
    c = lax.axis_index("c")
    s = lax.axis_index("s")
    w = s * NC + c
    _fill_flat(ones_v, SUB, 1.0)
    _fill_flat(zbuf, ZCH, 0.0)
    for k in range(RPT // ZCH):
        pltpu.sync_copy(zbuf, acc.at[pl.ds(s * RPT + k * ZCH, ZCH)])
    plsc.subcore_barrier()

    base_win = 48 * w + jnp.minimum(w, 26)
    nwin = jnp.where(w < 26, 49, 48)

    def window(i, carry):
        base = (base_win + i) * JW
        pltpu.sync_copy(ei_hbm.at[1, pl.ds(base, JW)], idxd)
        descs = [
            pltpu.async_copy(ones_v, acc.at[idxd.at[j]], sem, add=True)
            for j in range(JW)
        ]
        for d in descs:
            d.wait()
        return carry

    lax.fori_loop(0, nwin, window, 0)

    @pl.when(w == 31)
    def _():
        pltpu.sync_copy(ei_hbm.at[1, pl.ds(ROWS - 4, 4)],
                        idxd.at[pl.ds(0, 4)])
        descs = [
            pltpu.async_copy(ones_v, acc.at[idxd.at[j]], sem, add=True)
            for j in range(4)
        ]
        for d in descs:
            d.wait()

    plsc.subcore_barrier()
    pltpu.sync_copy(acc.at[pl.ds(s * RPT, RPT)],
                    out_hbm.at[c, pl.ds(s * RPT, RPT)])


@functools.cache
def _deg_call():
    return pl.kernel(
        _deg_body,
        out_type=jax.ShapeDtypeStruct((NC, NPAD), jnp.float32),
        mesh=_mesh(),
        compiler_params=pltpu.CompilerParams(use_tc_tiling_on_sc=False),
        scratch_types=[
            pltpu.VMEM((JW, SUB), jnp.int32),
            pltpu.VMEM((SUB,), jnp.float32),
            pltpu.VMEM((ZCH,), jnp.float32),
            pltpu.VMEM_SHARED((NPAD,), jnp.float32),
            pltpu.SemaphoreType.DMA,
        ],
    )


def _scatter_window(tab, ei_hbm, idxs, idxd, rows, acc, sem_g, sem_s, base,
                    nj):
    pltpu.sync_copy(ei_hbm.at[0, pl.ds(base, nj)], idxs.at[pl.ds(0, nj)])
    pltpu.sync_copy(ei_hbm.at[1, pl.ds(base, nj)], idxd.at[pl.ds(0, nj)])
    descs = [
        pltpu.async_copy(tab.at[idxs.at[j]],
                         rows.at[pl.ds(j * SUB, SUB)], sem_g)
        for j in range(nj)
    ]
    for d in descs:
        d.wait()
    descs = [
        pltpu.async_copy(rows.at[pl.ds(j * SUB, SUB)],
                         acc.at[idxd.at[j]], sem_s, add=True)
        for j in range(nj)
    ]
    for d in descs:
        d.wait()


def _run_shard(tab, ei_hbm, idxs, idxd, rows, acc, sem_g, sem_s, base_win,
               nwin, take_tail):
    def window(i, carry):
        _scatter_window(tab, ei_hbm, idxs, idxd, rows, acc, sem_g, sem_s,
                        (base_win + i) * JW, JW)
        return carry

    lax.fori_loop(0, nwin, window, 0)

    @pl.when(take_tail)
    def _():
        _scatter_window(tab, ei_hbm, idxs, idxd, rows, acc, sem_g, sem_s,
                        ROWS - 4, 4)


def _pass_scratch():
    return [
        pltpu.VMEM((JW, SUB), jnp.int32),
        pltpu.VMEM((JW, SUB), jnp.int32),
        pltpu.VMEM((JW * SUB, 16), jnp.float32),
        pltpu.VMEM_SHARED((NPAD, 16), jnp.float32),
        pltpu.SemaphoreType.DMA,
        pltpu.SemaphoreType.DMA,
    ]


def _edge_body(y, ei_hbm, out_hbm, idxs, idxd, rows, acc, sem_g, sem_s):
    c = lax.axis_index("c")
    s = lax.axis_index("s")
    _zero_acc_rows(rows, acc, s)
    plsc.subcore_barrier()
    w = s * NC + c
    _run_shard(y, ei_hbm, idxs, idxd, rows, acc, sem_g, sem_s,
               48 * w + jnp.minimum(w, 26), jnp.where(w < 26, 49, 48),
               w == 31)
    plsc.subcore_barrier()
    pltpu.sync_copy(acc.at[pl.ds(s * RPT, RPT)],
                    out_hbm.at[c, pl.ds(s * RPT, RPT)])


def _cols_body(ya, yb, ei_hbm, out_hbm, idxs, idxd, rows, acc, sem_g, sem_s):
    c = lax.axis_index("c")
    s = lax.axis_index("s")
    _zero_acc_rows(rows, acc, s)
    plsc.subcore_barrier()
    base_win = 97 * s + jnp.minimum(s, 10)
    nwin = jnp.where(s < 10, 98, 97)

    @pl.when(c == 0)
    def _():
        _run_shard(ya, ei_hbm, idxs, idxd, rows, acc, sem_g, sem_s,
                   base_win, nwin, s == 15)

    @pl.when(c == 1)
    def _():
        _run_shard(yb, ei_hbm, idxs, idxd, rows, acc, sem_g, sem_s,
                   base_win, nwin, s == 15)

    plsc.subcore_barrier()
    pltpu.sync_copy(acc.at[pl.ds(s * RPT, RPT)],
                    out_hbm.at[c, pl.ds(s * RPT, RPT)])


@functools.cache
def _pass_edge():
    return pl.kernel(
        _edge_body,
        out_type=jax.ShapeDtypeStruct((NC, NPAD, 16), jnp.float32),
        mesh=_mesh(),
        compiler_params=pltpu.CompilerParams(use_tc_tiling_on_sc=False),
        scratch_types=_pass_scratch(),
    )


@functools.cache
def _pass_cols():
    return pl.kernel(
        _cols_body,
        out_type=jax.ShapeDtypeStruct((NC, NPAD, 16), jnp.float32),
        mesh=_mesh(),
        compiler_params=pltpu.CompilerParams(use_tc_tiling_on_sc=False),
        scratch_types=_pass_scratch(),
    )


def _make_comb(emit_y):
    def body(*args):
        if emit_y:
            (s_hbm, zp_hbm, dv_hbm, dg_hbm, cv_hbm, z_hbm, y_hbm,
             s0c, s1c, zpc, dvc, dgc, cbuf, sem) = args
        else:
            (s_hbm, zp_hbm, dv_hbm, dg_hbm, cv_hbm, z_hbm,
             s0c, s1c, zpc, dvc, dgc, cbuf, sem) = args
        c = lax.axis_index("c")
        s = lax.axis_index("s")
        w = s * NC + c
        pltpu.sync_copy(cv_hbm, cbuf)
        cv = cbuf[...]
        for k in range(CPT // ZCH):
            rb = w * CPT + k * ZCH
            descs = [
                pltpu.async_copy(s_hbm.at[0, pl.ds(rb, ZCH)], s0c, sem),
                pltpu.async_copy(s_hbm.at[1, pl.ds(rb, ZCH)], s1c, sem),
                pltpu.async_copy(zp_hbm.at[pl.ds(rb, ZCH)], zpc, sem),
                pltpu.async_copy(dv_hbm.at[pl.ds(rb, ZCH)], dvc, sem),
                pltpu.async_copy(dg_hbm.at[pl.ds(rb, ZCH)], dgc, sem),
            ]
            for d in descs:
                d.wait()

            def row(i, carry):
                dv = dvc[i, :]
                vz = dv * (s0c[i, :] + s1c[i, :]) \
                    + dgc[i, :] * zpc[i, :] + cv
                s0c[i, :] = vz
                if emit_y:
                    s1c[i, :] = dv * vz
                return carry

            lax.fori_loop(0, ZCH, row, 0)
            pltpu.sync_copy(s0c, z_hbm.at[pl.ds(rb, ZCH)])
            if emit_y:
                pltpu.sync_copy(s1c, y_hbm.at[pl.ds(rb, ZCH)])

    n_out = 2 if emit_y else 1
    return pl.kernel(
        body,
        out_type=[jax.ShapeDtypeStruct((NPAD, 16), jnp.float32)] * n_out,
        mesh=_mesh(),
        compiler_params=pltpu.CompilerParams(use_tc_tiling_on_sc=False),
        scratch_types=[
            pltpu.VMEM((ZCH, 16), jnp.float32),
            pltpu.VMEM((ZCH, 16), jnp.float32),
            pltpu.VMEM((ZCH, 16), jnp.float32),
            pltpu.VMEM((ZCH, 16), jnp.float32),
            pltpu.VMEM((ZCH, 16), jnp.float32),
            pltpu.VMEM((16,), jnp.float32),
            pltpu.SemaphoreType.DMA,
        ],
    )


_comb_y = functools.cache(lambda: _make_comb(True))
_comb_final = functools.cache(lambda: _make_comb(False))


def _prep_body(dcnt_ref, x_ref, dv_ref, dg_ref, ya_ref, yb_ref):
    deg = dcnt_ref[...][:, 0] + dcnt_ref[...][:, 1] + 1.0
    dinv = lax.rsqrt(deg)
    dv_ref[...] = jnp.broadcast_to(dinv[:, None], (BLK, 16))
    dg_ref[...] = jnp.broadcast_to((1.0 / deg)[:, None], (BLK, 16))
    y = x_ref[...] * dinv[:, None]
    ya_ref[...] = y[:, :16]
    yb_ref[...] = y[:, 16:32]


def _prep_call(dcnt_t, xpad):
    return pl.pallas_call(
        _prep_body,
        grid=(GRID,),
        in_specs=[
            pl.BlockSpec((BLK, 2), lambda i: (i, 0)),
            pl.BlockSpec((BLK, 32), lambda i: (i, 0)),
        ],
        out_specs=[
            pl.BlockSpec((BLK, 16), lambda i: (i, 0)),
            pl.BlockSpec((BLK, 16), lambda i: (i, 0)),
            pl.BlockSpec((BLK, 16), lambda i: (i, 0)),
            pl.BlockSpec((BLK, 16), lambda i: (i, 0)),
        ],
        out_shape=[
            jax.ShapeDtypeStruct((NPAD, 16), jnp.float32),
            jax.ShapeDtypeStruct((NPAD, 16), jnp.float32),
            jax.ShapeDtypeStruct((NPAD, 16), jnp.float32),
            jax.ShapeDtypeStruct((NPAD, 16), jnp.float32),
        ],
    )(dcnt_t, xpad)


def _dense1_body(s_ref, x_ref, dv_ref, dg_ref, w1_ref, b1_ref, w2_ref,
                 w3_ref, w4_ref, b2_ref, b3_ref, g_ref, yn_ref, cvec_ref):
    dinv = dv_ref[...][:, :1]
    q = dinv * jnp.concatenate([s_ref[0], s_ref[1]], axis=1) \
        + dg_ref[...][:, :1] * x_ref[...]
    h1 = jnp.maximum(
        jnp.dot(q, w1_ref[...], preferred_element_type=jnp.float32)
        + b1_ref[...], 0.0)
    c34 = jnp.dot(w3_ref[...], w4_ref[...], preferred_element_type=jnp.float32)
    cmat = jnp.dot(w2_ref[...], c34, preferred_element_type=jnp.float32)
    g = jnp.dot(h1, cmat, preferred_element_type=jnp.float32)
    g_ref[...] = g
    yn_ref[...] = dinv * g
    cvec_ref[...] = jnp.concatenate(
        [jnp.dot(b2_ref[...], c34, preferred_element_type=jnp.float32),
         jnp.dot(b3_ref[...], w4_ref[...], preferred_element_type=jnp.float32)],
        axis=0)


def _dense1_call(s_ab, xpad, dv16, dg16, w1p, b1r, w2, w3, w4, b2r, b3r):
    full = lambda a: pl.BlockSpec(a.shape, lambda i: tuple(0 for _ in a.shape))
    return pl.pallas_call(
        _dense1_body,
        grid=(GRID,),
        in_specs=[
            pl.BlockSpec((NC, BLK, 16), lambda i: (0, i, 0)),
            pl.BlockSpec((BLK, 32), lambda i: (i, 0)),
            pl.BlockSpec((BLK, 16), lambda i: (i, 0)),
            pl.BlockSpec((BLK, 16), lambda i: (i, 0)),
            full(w1p), full(b1r), full(w2), full(w3), full(w4),
            full(b2r), full(b3r),
        ],
        out_specs=[
            pl.BlockSpec((BLK, 16), lambda i: (i, 0)),
            pl.BlockSpec((BLK, 16), lambda i: (i, 0)),
            pl.BlockSpec((2, 16), lambda i: (0, 0)),
        ],
        out_shape=[
            jax.ShapeDtypeStruct((NPAD, 16), jnp.float32),
            jax.ShapeDtypeStruct((NPAD, 16), jnp.float32),
            jax.ShapeDtypeStruct((2, 16), jnp.float32),
        ],
    )(s_ab, xpad, dv16, dg16, w1p, b1r, w2, w3, w4, b2r, b3r)


def kernel(x, edge_index, W1, b1, W2, b2, W3, b3, W4, b4):
    ei3 = edge_index.astype(jnp.int32).reshape(2, ROWS, SUB)
    xpad = jnp.zeros((NPAD, 32), jnp.float32).at[:N, :29].set(x)
    w1p = jnp.zeros((32, 96), jnp.float32).at[:29].set(W1)

    dcnt = _deg_call()(ei3)
    dv16, dg16, ya, yb = _prep_call(dcnt.T, xpad)
    s_ab = _pass_cols()(ya, yb, ei3)
    g, yn, cvec = _dense1_call(s_ab, xpad, dv16, dg16, w1p, b1[None], W2,
                               W3, W4, b2[None], b3[None])
    s2 = _pass_edge()(yn, ei3)
    z2, y2 = _comb_y()(s2, g, dv16, dg16, cvec[0])
    s3 = _pass_edge()(y2, ei3)
    z3, y3 = _comb_y()(s3, z2, dv16, dg16, cvec[1])
    s4 = _pass_edge()(y3, ei3)
    (h4,) = _comb_final()(s4, z3, dv16, dg16, b4)
    return h4[:N]


# speedup vs baseline: 37.3307x; 1.0992x over previous
import functools

import jax
import jax.numpy as jnp
from jax import lax
from jax.experimental import pallas as pl
from jax.experimental.pallas import tpu as pltpu
from jax.experimental.pallas import tpu_sc as plsc

N = 100000
E = 1600000
NPAD = 100352
ROWS = E // 128
SUB = 128
JW = 6
JD = 8
NC = 2
NS = 16
RPT = NPAD // NS
ZCH = 392
CPT = NPAD // 32
BLK = 2048
GRID = NPAD // BLK


@functools.cache
def _mesh():
    return plsc.VectorSubcoreMesh(core_axis_name="c", subcore_axis_name="s",
                                  num_cores=NC, num_subcores=NS)


def _fill_rows(ref, nrows, val):
    def body(i, carry):
        ref[i, :] = jnp.full((16,), val, jnp.float32)
        return carry
    lax.fori_loop(0, nrows, body, 0)


def _fill_flat(ref, n, val):
    def body(i, carry):
        ref[pl.ds(i * 16, 16)] = jnp.full((16,), val, jnp.float32)
        return carry
    lax.fori_loop(0, n // 16, body, 0)


def _zero_acc_rows(rows, acc, s):
    _fill_rows(rows, ZCH, 0.0)
    for k in range(RPT // ZCH):
        pltpu.sync_copy(rows.at[pl.ds(0, ZCH)],
                        acc.at[pl.ds(s * RPT + k * ZCH, ZCH)])


def _win_gather(tab, ei_hbm, idxs, idxd, rows, sem, base):
    pltpu.sync_copy(ei_hbm.at[0, pl.ds(base, JW)], idxs.at[pl.ds(0, JW)])
    pltpu.sync_copy(ei_hbm.at[1, pl.ds(base, JW)], idxd.at[pl.ds(0, JW)])
    return [
        pltpu.async_copy(tab.at[idxs.at[j]],
                         rows.at[pl.ds(j * SUB, SUB)], sem)
        for j in range(JW)
    ]


def _win_scatter(rows, idxd, acc, sem):
    return [
        pltpu.async_copy(rows.at[pl.ds(j * SUB, SUB)],
                         acc.at[idxd.at[j]], sem, add=True)
        for j in range(JW)
    ]


def _win_pair(tab, ei_hbm, bufa, bufb, acc, base_a, base_b):
    isa, ida, rwa, sga, ssa = bufa
    isb, idb, rwb, sgb, ssb = bufb
    ga = _win_gather(tab, ei_hbm, isa, ida, rwa, sga, base_a)
    gb = _win_gather(tab, ei_hbm, isb, idb, rwb, sgb, base_b)
    for d in ga:
        d.wait()
    sa = _win_scatter(rwa, ida, acc, ssa)
    for d in gb:
        d.wait()
    sb = _win_scatter(rwb, idb, acc, ssb)
    for d in sa:
        d.wait()
    for d in sb:
        d.wait()


def _win_single(tab, ei_hbm, bufa, acc, base):
    isa, ida, rwa, sga, ssa = bufa
    for d in _win_gather(tab, ei_hbm, isa, ida, rwa, sga, base):
        d.wait()
    for d in _win_scatter(rwa, ida, acc, ssa):
        d.wait()


def _row_single(tab, ei_hbm, bufa, acc, r):
    isa, ida, rwa, sga, ssa = bufa
    pltpu.sync_copy(ei_hbm.at[0, pl.ds(r, 1)], isa.at[pl.ds(0, 1)])
    pltpu.sync_copy(ei_hbm.at[1, pl.ds(r, 1)], ida.at[pl.ds(0, 1)])
    pltpu.async_copy(tab.at[isa.at[0]], rwa.at[pl.ds(0, SUB)], sga).wait()
    pltpu.async_copy(rwa.at[pl.ds(0, SUB)], acc.at[ida.at[0]], ssa,
                     add=True).wait()


def _deg_body(ei_hbm, out_hbm, idxd, ones_v, zbuf, acc, sem):
    c = lax.axis_index("c")
    s = lax.axis_index("s")
    w = s * NC + c
    _fill_flat(ones_v, SUB, 1.0)
    _fill_flat(zbuf, 400, 0.0)
    for k in range(RPT // ZCH):
        pltpu.sync_copy(zbuf.at[pl.ds(0, ZCH)],
                        acc.at[pl.ds(s * RPT + k * ZCH, ZCH)])
    plsc.subcore_barrier()

    base_win = 48 * w + jnp.minimum(w, 26)
    nwin = jnp.where(w < 26, 49, 48)

    def window(i, carry):
        base = (base_win + i) * JD
        pltpu.sync_copy(ei_hbm.at[1, pl.ds(base, JD)], idxd)
        descs = [
            pltpu.async_copy(ones_v, acc.at[idxd.at[j]], sem, add=True)
            for j in range(JD)
        ]
        for d in descs:
            d.wait()
        return carry

    lax.fori_loop(0, nwin, window, 0)

    @pl.when(w == 31)
    def _():
        pltpu.sync_copy(ei_hbm.at[1, pl.ds(ROWS - 4, 4)],
                        idxd.at[pl.ds(0, 4)])
        descs = [
            pltpu.async_copy(ones_v, acc.at[idxd.at[j]], sem, add=True)
            for j in range(4)
        ]
        for d in descs:
            d.wait()

    plsc.subcore_barrier()
    pltpu.sync_copy(acc.at[pl.ds(s * RPT, RPT)],
                    out_hbm.at[c, pl.ds(s * RPT, RPT)])


@functools.cache
def _deg_call():
    return pl.kernel(
        _deg_body,
        out_type=jax.ShapeDtypeStruct((NC, NPAD), jnp.float32),
        mesh=_mesh(),
        compiler_params=pltpu.CompilerParams(use_tc_tiling_on_sc=False),
        scratch_types=[
            pltpu.VMEM((JD, SUB), jnp.int32),
            pltpu.VMEM((SUB,), jnp.float32),
            pltpu.VMEM((400,), jnp.float32),
            pltpu.VMEM_SHARED((NPAD,), jnp.float32),
            pltpu.SemaphoreType.DMA,
        ],
    )


def _pass_scratch():
    buf = [
        pltpu.VMEM((8, SUB), jnp.int32),
        pltpu.VMEM((8, SUB), jnp.int32),
        pltpu.VMEM((JW * SUB, 16), jnp.float32),
        pltpu.SemaphoreType.DMA,
        pltpu.SemaphoreType.DMA,
    ]
    return buf + buf + [pltpu.VMEM_SHARED((NPAD, 16), jnp.float32)]


def _edge_body(y, ei_hbm, out_hbm, isa, ida, rwa, sga, ssa, isb, idb, rwb,
               sgb, ssb, acc):
    c = lax.axis_index("c")
    s = lax.axis_index("s")
    bufa = (isa, ida, rwa, sga, ssa)
    bufb = (isb, idb, rwb, sgb, ssb)
    _zero_acc_rows(rwa, acc, s)
    plsc.subcore_barrier()
    w = s * NC + c
    base0 = w * 390

    def pair(i, carry):
        _win_pair(y, ei_hbm, bufa, bufb, acc,
                  base0 + 12 * i, base0 + 12 * i + JW)
        return carry

    lax.fori_loop(0, 32, pair, 0)
    _win_single(y, ei_hbm, bufa, acc, base0 + 384)

    @pl.when(w < 20)
    def _():
        _row_single(y, ei_hbm, bufa, acc, 12480 + w)

    plsc.subcore_barrier()
    pltpu.sync_copy(acc.at[pl.ds(s * RPT, RPT)],
                    out_hbm.at[c, pl.ds(s * RPT, RPT)])


def _cols_shard(tab, ei_hbm, bufa, bufb, acc, s):
    base0 = s * 780

    def pair(i, carry):
        _win_pair(tab, ei_hbm, bufa, bufb, acc,
                  base0 + 12 * i, base0 + 12 * i + JW)
        return carry

    lax.fori_loop(0, 65, pair, 0)
    _row_single(tab, ei_hbm, bufa, acc, 12480 + s)

    @pl.when(s < 4)
    def _():
        _row_single(tab, ei_hbm, bufa, acc, 12496 + s)


def _cols_body(ya, yb, ei_hbm, out_hbm, isa, ida, rwa, sga, ssa, isb, idb,
               rwb, sgb, ssb, acc):
    c = lax.axis_index("c")
    s = lax.axis_index("s")
    bufa = (isa, ida, rwa, sga, ssa)
    bufb = (isb, idb, rwb, sgb, ssb)
    _zero_acc_rows(rwa, acc, s)
    plsc.subcore_barrier()

    @pl.when(c == 0)
    def _():
        _cols_shard(ya, ei_hbm, bufa, bufb, acc, s)

    @pl.when(c == 1)
    def _():
        _cols_shard(yb, ei_hbm, bufa, bufb, acc, s)

    plsc.subcore_barrier()
    pltpu.sync_copy(acc.at[pl.ds(s * RPT, RPT)],
                    out_hbm.at[c, pl.ds(s * RPT, RPT)])


@functools.cache
def _pass_edge():
    return pl.kernel(
        _edge_body,
        out_type=jax.ShapeDtypeStruct((NC, NPAD, 16), jnp.float32),
        mesh=_mesh(),
        compiler_params=pltpu.CompilerParams(use_tc_tiling_on_sc=False),
        scratch_types=_pass_scratch(),
    )


@functools.cache
def _pass_cols():
    return pl.kernel(
        _cols_body,
        out_type=jax.ShapeDtypeStruct((NC, NPAD, 16), jnp.float32),
        mesh=_mesh(),
        compiler_params=pltpu.CompilerParams(use_tc_tiling_on_sc=False),
        scratch_types=_pass_scratch(),
    )


def _make_comb(emit_y):
    def body(*args):
        if emit_y:
            (s_hbm, zp_hbm, dv_hbm, dg_hbm, cv_hbm, z_hbm, y_hbm,
             s0c, s1c, zpc, dvc, dgc, cbuf, sem) = args
        else:
            (s_hbm, zp_hbm, dv_hbm, dg_hbm, cv_hbm, z_hbm,
             s0c, s1c, zpc, dvc, dgc, cbuf, sem) = args
        c = lax.axis_index("c")
        s = lax.axis_index("s")
        w = s * NC + c
        pltpu.sync_copy(cv_hbm, cbuf)
        cv = cbuf[...]
        for k in range(CPT // ZCH):
            rb = w * CPT + k * ZCH
            descs = [
                pltpu.async_copy(s_hbm.at[0, pl.ds(rb, ZCH)], s0c, sem),
                pltpu.async_copy(s_hbm.at[1, pl.ds(rb, ZCH)], s1c, sem),
                pltpu.async_copy(zp_hbm.at[pl.ds(rb, ZCH)], zpc, sem),
                pltpu.async_copy(dv_hbm.at[pl.ds(rb, ZCH)], dvc, sem),
                pltpu.async_copy(dg_hbm.at[pl.ds(rb, ZCH)], dgc, sem),
            ]
            for d in descs:
                d.wait()

            def row(i, carry):
                dv = dvc[i, :]
                vz = dv * (s0c[i, :] + s1c[i, :]) \
                    + dgc[i, :] * zpc[i, :] + cv
                s0c[i, :] = vz
                if emit_y:
                    s1c[i, :] = dv * vz
                return carry

            lax.fori_loop(0, ZCH, row, 0)
            pltpu.sync_copy(s0c, z_hbm.at[pl.ds(rb, ZCH)])
            if emit_y:
                pltpu.sync_copy(s1c, y_hbm.at[pl.ds(rb, ZCH)])

    n_out = 2 if emit_y else 1
    return pl.kernel(
        body,
        out_type=[jax.ShapeDtypeStruct((NPAD, 16), jnp.float32)] * n_out,
        mesh=_mesh(),
        compiler_params=pltpu.CompilerParams(use_tc_tiling_on_sc=False),
        scratch_types=[
            pltpu.VMEM((ZCH, 16), jnp.float32),
            pltpu.VMEM((ZCH, 16), jnp.float32),
            pltpu.VMEM((ZCH, 16), jnp.float32),
            pltpu.VMEM((ZCH, 16), jnp.float32),
            pltpu.VMEM((ZCH, 16), jnp.float32),
            pltpu.VMEM((16,), jnp.float32),
            pltpu.SemaphoreType.DMA,
        ],
    )


_comb_y = functools.cache(lambda: _make_comb(True))
_comb_final = functools.cache(lambda: _make_comb(False))


def _prep_body(dcnt_ref, x_ref, dv_ref, dg_ref, ya_ref, yb_ref):
    deg = dcnt_ref[...][:, 0] + dcnt_ref[...][:, 1] + 1.0
    dinv = lax.rsqrt(deg)
    dv_ref[...] = jnp.broadcast_to(dinv[:, None], (BLK, 16))
    dg_ref[...] = jnp.broadcast_to((1.0 / deg)[:, None], (BLK, 16))
    y = x_ref[...] * dinv[:, None]
    ya_ref[...] = y[:, :16]
    yb_ref[...] = y[:, 16:32]


def _prep_call(dcnt_t, xpad):
    return pl.pallas_call(
        _prep_body,
        grid=(GRID,),
        in_specs=[
            pl.BlockSpec((BLK, 2), lambda i: (i, 0)),
            pl.BlockSpec((BLK, 32), lambda i: (i, 0)),
        ],
        out_specs=[
            pl.BlockSpec((BLK, 16), lambda i: (i, 0)),
            pl.BlockSpec((BLK, 16), lambda i: (i, 0)),
            pl.BlockSpec((BLK, 16), lambda i: (i, 0)),
            pl.BlockSpec((BLK, 16), lambda i: (i, 0)),
        ],
        out_shape=[
            jax.ShapeDtypeStruct((NPAD, 16), jnp.float32),
            jax.ShapeDtypeStruct((NPAD, 16), jnp.float32),
            jax.ShapeDtypeStruct((NPAD, 16), jnp.float32),
            jax.ShapeDtypeStruct((NPAD, 16), jnp.float32),
        ],
    )(dcnt_t, xpad)


def _dense1_body(s_ref, x_ref, dv_ref, dg_ref, w1_ref, b1_ref, w2_ref,
                 w3_ref, w4_ref, b2_ref, b3_ref, g_ref, yn_ref, cvec_ref):
    dinv = dv_ref[...][:, :1]
    q = dinv * jnp.concatenate([s_ref[0], s_ref[1]], axis=1) \
        + dg_ref[...][:, :1] * x_ref[...]
    h1 = jnp.maximum(
        jnp.dot(q, w1_ref[...], preferred_element_type=jnp.float32)
        + b1_ref[...], 0.0)
    c34 = jnp.dot(w3_ref[...], w4_ref[...], preferred_element_type=jnp.float32)
    cmat = jnp.dot(w2_ref[...], c34, preferred_element_type=jnp.float32)
    g = jnp.dot(h1, cmat, preferred_element_type=jnp.float32)
    g_ref[...] = g
    yn_ref[...] = dinv * g
    cvec_ref[...] = jnp.concatenate(
        [jnp.dot(b2_ref[...], c34, preferred_element_type=jnp.float32),
         jnp.dot(b3_ref[...], w4_ref[...], preferred_element_type=jnp.float32)],
        axis=0)


def _dense1_call(s_ab, xpad, dv16, dg16, w1p, b1r, w2, w3, w4, b2r, b3r):
    full = lambda a: pl.BlockSpec(a.shape, lambda i: tuple(0 for _ in a.shape))
    return pl.pallas_call(
        _dense1_body,
        grid=(GRID,),
        in_specs=[
            pl.BlockSpec((NC, BLK, 16), lambda i: (0, i, 0)),
            pl.BlockSpec((BLK, 32), lambda i: (i, 0)),
            pl.BlockSpec((BLK, 16), lambda i: (i, 0)),
            pl.BlockSpec((BLK, 16), lambda i: (i, 0)),
            full(w1p), full(b1r), full(w2), full(w3), full(w4),
            full(b2r), full(b3r),
        ],
        out_specs=[
            pl.BlockSpec((BLK, 16), lambda i: (i, 0)),
            pl.BlockSpec((BLK, 16), lambda i: (i, 0)),
            pl.BlockSpec((2, 16), lambda i: (0, 0)),
        ],
        out_shape=[
            jax.ShapeDtypeStruct((NPAD, 16), jnp.float32),
            jax.ShapeDtypeStruct((NPAD, 16), jnp.float32),
            jax.ShapeDtypeStruct((2, 16), jnp.float32),
        ],
    )(s_ab, xpad, dv16, dg16, w1p, b1r, w2, w3, w4, b2r, b3r)


def kernel(x, edge_index, W1, b1, W2, b2, W3, b3, W4, b4):
    ei3 = edge_index.astype(jnp.int32).reshape(2, ROWS, SUB)
    xpad = jnp.zeros((NPAD, 32), jnp.float32).at[:N, :29].set(x)
    w1p = jnp.zeros((32, 96), jnp.float32).at[:29].set(W1)

    dcnt = _deg_call()(ei3)
    dv16, dg16, ya, yb = _prep_call(dcnt.T, xpad)
    s_ab = _pass_cols()(ya, yb, ei3)
    g, yn, cvec = _dense1_call(s_ab, xpad, dv16, dg16, w1p, b1[None], W2,
                               W3, W4, b2[None], b3[None])
    s2 = _pass_edge()(yn, ei3)
    z2, y2 = _comb_y()(s2, g, dv16, dg16, cvec[0])
    s3 = _pass_edge()(y2, ei3)
    z3, y3 = _comb_y()(s3, z2, dv16, dg16, cvec[1])
    s4 = _pass_edge()(y3, ei3)
    (h4,) = _comb_final()(s4, z3, dv16, dg16, b4)
    return h4[:N]


# speedup vs baseline: 37.4050x; 1.0020x over previous
import functools

import jax
import jax.numpy as jnp
from jax import lax
from jax.experimental import pallas as pl
from jax.experimental.pallas import tpu as pltpu
from jax.experimental.pallas import tpu_sc as plsc

N = 100000
E = 1600000
NPAD = 100352
ROWS = E // 128
SUB = 128
JW = 6
JD = 8
NC = 2
NS = 16
RPT = NPAD // NS
ZCH = 392
CPT = NPAD // 32
BLK = 2048
GRID = NPAD // BLK


@functools.cache
def _mesh():
    return plsc.VectorSubcoreMesh(core_axis_name="c", subcore_axis_name="s",
                                  num_cores=NC, num_subcores=NS)


def _fill_rows(ref, nrows, val):
    def body(i, carry):
        ref[i, :] = jnp.full((16,), val, jnp.float32)
        return carry
    lax.fori_loop(0, nrows, body, 0)


def _fill_flat(ref, n, val):
    def body(i, carry):
        ref[pl.ds(i * 16, 16)] = jnp.full((16,), val, jnp.float32)
        return carry
    lax.fori_loop(0, n // 16, body, 0)


def _zero_acc_rows(rows, acc, s):
    _fill_rows(rows, ZCH, 0.0)
    for k in range(RPT // ZCH):
        pltpu.sync_copy(rows.at[pl.ds(0, ZCH)],
                        acc.at[pl.ds(s * RPT + k * ZCH, ZCH)])


def _win_gather(tab, ei_hbm, idxs, idxd, rows, sem, base):
    pltpu.sync_copy(ei_hbm.at[0, pl.ds(base * SUB, JW * SUB)], idxs)
    pltpu.sync_copy(ei_hbm.at[1, pl.ds(base * SUB, JW * SUB)], idxd)
    return [pltpu.async_copy(tab.at[idxs], rows, sem)]


def _win_scatter(rows, idxd, acc, sem):
    return [pltpu.async_copy(rows, acc.at[idxd], sem, add=True)]


def _win_pair(tab, ei_hbm, bufa, bufb, acc, base_a, base_b):
    isa, ida, rwa, sga, ssa = bufa
    isb, idb, rwb, sgb, ssb = bufb
    ga = _win_gather(tab, ei_hbm, isa, ida, rwa, sga, base_a)
    gb = _win_gather(tab, ei_hbm, isb, idb, rwb, sgb, base_b)
    for d in ga:
        d.wait()
    sa = _win_scatter(rwa, ida, acc, ssa)
    for d in gb:
        d.wait()
    sb = _win_scatter(rwb, idb, acc, ssb)
    for d in sa:
        d.wait()
    for d in sb:
        d.wait()


def _win_single(tab, ei_hbm, bufa, acc, base):
    isa, ida, rwa, sga, ssa = bufa
    for d in _win_gather(tab, ei_hbm, isa, ida, rwa, sga, base):
        d.wait()
    for d in _win_scatter(rwa, ida, acc, ssa):
        d.wait()


def _row_single(tab, ei_hbm, bufa, acc, r):
    isa, ida, rwa, sga, ssa = bufa
    pltpu.sync_copy(ei_hbm.at[0, pl.ds(r * SUB, SUB)], isa.at[pl.ds(0, SUB)])
    pltpu.sync_copy(ei_hbm.at[1, pl.ds(r * SUB, SUB)], ida.at[pl.ds(0, SUB)])
    pltpu.async_copy(tab.at[isa.at[pl.ds(0, SUB)]], rwa.at[pl.ds(0, SUB)],
                     sga).wait()
    pltpu.async_copy(rwa.at[pl.ds(0, SUB)], acc.at[ida.at[pl.ds(0, SUB)]],
                     ssa, add=True).wait()


def _deg_body(ei_hbm, out_hbm, idxd, ones_v, zbuf, acc, sem):
    c = lax.axis_index("c")
    s = lax.axis_index("s")
    w = s * NC + c
    _fill_flat(ones_v, JD * SUB, 1.0)
    _fill_flat(zbuf, 400, 0.0)
    for k in range(RPT // ZCH):
        pltpu.sync_copy(zbuf.at[pl.ds(0, ZCH)],
                        acc.at[pl.ds(s * RPT + k * ZCH, ZCH)])
    plsc.subcore_barrier()

    base_win = 48 * w + jnp.minimum(w, 26)
    nwin = jnp.where(w < 26, 49, 48)

    def window(i, carry):
        base = (base_win + i) * JD * SUB
        pltpu.sync_copy(ei_hbm.at[1, pl.ds(base, JD * SUB)], idxd)
        pltpu.async_copy(ones_v, acc.at[idxd], sem, add=True).wait()
        return carry

    lax.fori_loop(0, nwin, window, 0)

    @pl.when(w == 31)
    def _():
        pltpu.sync_copy(ei_hbm.at[1, pl.ds((ROWS - 4) * SUB, 4 * SUB)],
                        idxd.at[pl.ds(0, 4 * SUB)])
        pltpu.async_copy(ones_v.at[pl.ds(0, 4 * SUB)],
                         acc.at[idxd.at[pl.ds(0, 4 * SUB)]], sem,
                         add=True).wait()

    plsc.subcore_barrier()
    pltpu.sync_copy(acc.at[pl.ds(s * RPT, RPT)],
                    out_hbm.at[c, pl.ds(s * RPT, RPT)])


@functools.cache
def _deg_call():
    return pl.kernel(
        _deg_body,
        out_type=jax.ShapeDtypeStruct((NC, NPAD), jnp.float32),
        mesh=_mesh(),
        compiler_params=pltpu.CompilerParams(use_tc_tiling_on_sc=False),
        scratch_types=[
            pltpu.VMEM((JD * SUB,), jnp.int32),
            pltpu.VMEM((JD * SUB,), jnp.float32),
            pltpu.VMEM((400,), jnp.float32),
            pltpu.VMEM_SHARED((NPAD,), jnp.float32),
            pltpu.SemaphoreType.DMA,
        ],
    )


def _pass_scratch():
    buf = [
        pltpu.VMEM((JW * SUB,), jnp.int32),
        pltpu.VMEM((JW * SUB,), jnp.int32),
        pltpu.VMEM((JW * SUB, 16), jnp.float32),
        pltpu.SemaphoreType.DMA,
        pltpu.SemaphoreType.DMA,
    ]
    return buf + buf + [pltpu.VMEM_SHARED((NPAD, 16), jnp.float32)]


def _edge_body(y, ei_hbm, out_hbm, isa, ida, rwa, sga, ssa, isb, idb, rwb,
               sgb, ssb, acc):
    c = lax.axis_index("c")
    s = lax.axis_index("s")
    bufa = (isa, ida, rwa, sga, ssa)
    bufb = (isb, idb, rwb, sgb, ssb)
    _zero_acc_rows(rwa, acc, s)
    plsc.subcore_barrier()
    w = s * NC + c
    base0 = w * 390

    def pair(i, carry):
        _win_pair(y, ei_hbm, bufa, bufb, acc,
                  base0 + 12 * i, base0 + 12 * i + JW)
        return carry

    lax.fori_loop(0, 32, pair, 0)
    _win_single(y, ei_hbm, bufa, acc, base0 + 384)

    @pl.when(w < 20)
    def _():
        _row_single(y, ei_hbm, bufa, acc, 12480 + w)

    plsc.subcore_barrier()
    pltpu.sync_copy(acc.at[pl.ds(s * RPT, RPT)],
                    out_hbm.at[c, pl.ds(s * RPT, RPT)])


def _cols_shard(tab, ei_hbm, bufa, bufb, acc, s):
    base0 = s * 780

    def pair(i, carry):
        _win_pair(tab, ei_hbm, bufa, bufb, acc,
                  base0 + 12 * i, base0 + 12 * i + JW)
        return carry

    lax.fori_loop(0, 65, pair, 0)
    _row_single(tab, ei_hbm, bufa, acc, 12480 + s)

    @pl.when(s < 4)
    def _():
        _row_single(tab, ei_hbm, bufa, acc, 12496 + s)


def _cols_body(ya, yb, ei_hbm, out_hbm, isa, ida, rwa, sga, ssa, isb, idb,
               rwb, sgb, ssb, acc):
    c = lax.axis_index("c")
    s = lax.axis_index("s")
    bufa = (isa, ida, rwa, sga, ssa)
    bufb = (isb, idb, rwb, sgb, ssb)
    _zero_acc_rows(rwa, acc, s)
    plsc.subcore_barrier()

    @pl.when(c == 0)
    def _():
        _cols_shard(ya, ei_hbm, bufa, bufb, acc, s)

    @pl.when(c == 1)
    def _():
        _cols_shard(yb, ei_hbm, bufa, bufb, acc, s)

    plsc.subcore_barrier()
    pltpu.sync_copy(acc.at[pl.ds(s * RPT, RPT)],
                    out_hbm.at[c, pl.ds(s * RPT, RPT)])


@functools.cache
def _pass_edge():
    return pl.kernel(
        _edge_body,
        out_type=jax.ShapeDtypeStruct((NC, NPAD, 16), jnp.float32),
        mesh=_mesh(),
        compiler_params=pltpu.CompilerParams(use_tc_tiling_on_sc=False),
        scratch_types=_pass_scratch(),
    )


@functools.cache
def _pass_cols():
    return pl.kernel(
        _cols_body,
        out_type=jax.ShapeDtypeStruct((NC, NPAD, 16), jnp.float32),
        mesh=_mesh(),
        compiler_params=pltpu.CompilerParams(use_tc_tiling_on_sc=False),
        scratch_types=_pass_scratch(),
    )


def _make_comb(emit_y):
    def body(*args):
        if emit_y:
            (s_hbm, zp_hbm, dv_hbm, dg_hbm, cv_hbm, z_hbm, y_hbm,
             s0c, s1c, zpc, dvc, dgc, cbuf, sem) = args
        else:
            (s_hbm, zp_hbm, dv_hbm, dg_hbm, cv_hbm, z_hbm,
             s0c, s1c, zpc, dvc, dgc, cbuf, sem) = args
        c = lax.axis_index("c")
        s = lax.axis_index("s")
        w = s * NC + c
        pltpu.sync_copy(cv_hbm, cbuf)
        cv = cbuf[...]
        for k in range(CPT // ZCH):
            rb = w * CPT + k * ZCH
            descs = [
                pltpu.async_copy(s_hbm.at[0, pl.ds(rb, ZCH)], s0c, sem),
                pltpu.async_copy(s_hbm.at[1, pl.ds(rb, ZCH)], s1c, sem),
                pltpu.async_copy(zp_hbm.at[pl.ds(rb, ZCH)], zpc, sem),
                pltpu.async_copy(dv_hbm.at[pl.ds(rb, ZCH)], dvc, sem),
                pltpu.async_copy(dg_hbm.at[pl.ds(rb, ZCH)], dgc, sem),
            ]
            for d in descs:
                d.wait()

            def row(i, carry):
                dv = dvc[i, :]
                vz = dv * (s0c[i, :] + s1c[i, :]) \
                    + dgc[i, :] * zpc[i, :] + cv
                s0c[i, :] = vz
                if emit_y:
                    s1c[i, :] = dv * vz
                return carry

            lax.fori_loop(0, ZCH, row, 0)
            pltpu.sync_copy(s0c, z_hbm.at[pl.ds(rb, ZCH)])
            if emit_y:
                pltpu.sync_copy(s1c, y_hbm.at[pl.ds(rb, ZCH)])

    n_out = 2 if emit_y else 1
    return pl.kernel(
        body,
        out_type=[jax.ShapeDtypeStruct((NPAD, 16), jnp.float32)] * n_out,
        mesh=_mesh(),
        compiler_params=pltpu.CompilerParams(use_tc_tiling_on_sc=False),
        scratch_types=[
            pltpu.VMEM((ZCH, 16), jnp.float32),
            pltpu.VMEM((ZCH, 16), jnp.float32),
            pltpu.VMEM((ZCH, 16), jnp.float32),
            pltpu.VMEM((ZCH, 16), jnp.float32),
            pltpu.VMEM((ZCH, 16), jnp.float32),
            pltpu.VMEM((16,), jnp.float32),
            pltpu.SemaphoreType.DMA,
        ],
    )


_comb_y = functools.cache(lambda: _make_comb(True))
_comb_final = functools.cache(lambda: _make_comb(False))


def _prep_body(dcnt_ref, x_ref, dv_ref, dg_ref, ya_ref, yb_ref):
    deg = dcnt_ref[...][:, 0] + dcnt_ref[...][:, 1] + 1.0
    dinv = lax.rsqrt(deg)
    dv_ref[...] = jnp.broadcast_to(dinv[:, None], (BLK, 16))
    dg_ref[...] = jnp.broadcast_to((1.0 / deg)[:, None], (BLK, 16))
    y = x_ref[...] * dinv[:, None]
    ya_ref[...] = y[:, :16]
    yb_ref[...] = y[:, 16:32]


def _prep_call(dcnt_t, xpad):
    return pl.pallas_call(
        _prep_body,
        grid=(GRID,),
        in_specs=[
            pl.BlockSpec((BLK, 2), lambda i: (i, 0)),
            pl.BlockSpec((BLK, 32), lambda i: (i, 0)),
        ],
        out_specs=[
            pl.BlockSpec((BLK, 16), lambda i: (i, 0)),
            pl.BlockSpec((BLK, 16), lambda i: (i, 0)),
            pl.BlockSpec((BLK, 16), lambda i: (i, 0)),
            pl.BlockSpec((BLK, 16), lambda i: (i, 0)),
        ],
        out_shape=[
            jax.ShapeDtypeStruct((NPAD, 16), jnp.float32),
            jax.ShapeDtypeStruct((NPAD, 16), jnp.float32),
            jax.ShapeDtypeStruct((NPAD, 16), jnp.float32),
            jax.ShapeDtypeStruct((NPAD, 16), jnp.float32),
        ],
    )(dcnt_t, xpad)


def _dense1_body(s_ref, x_ref, dv_ref, dg_ref, w1_ref, b1_ref, w2_ref,
                 w3_ref, w4_ref, b2_ref, b3_ref, g_ref, yn_ref, cvec_ref):
    dinv = dv_ref[...][:, :1]
    q = dinv * jnp.concatenate([s_ref[0], s_ref[1]], axis=1) \
        + dg_ref[...][:, :1] * x_ref[...]
    h1 = jnp.maximum(
        jnp.dot(q, w1_ref[...], preferred_element_type=jnp.float32)
        + b1_ref[...], 0.0)
    c34 = jnp.dot(w3_ref[...], w4_ref[...], preferred_element_type=jnp.float32)
    cmat = jnp.dot(w2_ref[...], c34, preferred_element_type=jnp.float32)
    g = jnp.dot(h1, cmat, preferred_element_type=jnp.float32)
    g_ref[...] = g
    yn_ref[...] = dinv * g
    cvec_ref[...] = jnp.concatenate(
        [jnp.dot(b2_ref[...], c34, preferred_element_type=jnp.float32),
         jnp.dot(b3_ref[...], w4_ref[...], preferred_element_type=jnp.float32)],
        axis=0)


def _dense1_call(s_ab, xpad, dv16, dg16, w1p, b1r, w2, w3, w4, b2r, b3r):
    full = lambda a: pl.BlockSpec(a.shape, lambda i: tuple(0 for _ in a.shape))
    return pl.pallas_call(
        _dense1_body,
        grid=(GRID,),
        in_specs=[
            pl.BlockSpec((NC, BLK, 16), lambda i: (0, i, 0)),
            pl.BlockSpec((BLK, 32), lambda i: (i, 0)),
            pl.BlockSpec((BLK, 16), lambda i: (i, 0)),
            pl.BlockSpec((BLK, 16), lambda i: (i, 0)),
            full(w1p), full(b1r), full(w2), full(w3), full(w4),
            full(b2r), full(b3r),
        ],
        out_specs=[
            pl.BlockSpec((BLK, 16), lambda i: (i, 0)),
            pl.BlockSpec((BLK, 16), lambda i: (i, 0)),
            pl.BlockSpec((2, 16), lambda i: (0, 0)),
        ],
        out_shape=[
            jax.ShapeDtypeStruct((NPAD, 16), jnp.float32),
            jax.ShapeDtypeStruct((NPAD, 16), jnp.float32),
            jax.ShapeDtypeStruct((2, 16), jnp.float32),
        ],
    )(s_ab, xpad, dv16, dg16, w1p, b1r, w2, w3, w4, b2r, b3r)


def kernel(x, edge_index, W1, b1, W2, b2, W3, b3, W4, b4):
    ei3 = edge_index.astype(jnp.int32)
    xpad = jnp.zeros((NPAD, 32), jnp.float32).at[:N, :29].set(x)
    w1p = jnp.zeros((32, 96), jnp.float32).at[:29].set(W1)

    dcnt = _deg_call()(ei3)
    dv16, dg16, ya, yb = _prep_call(dcnt.T, xpad)
    s_ab = _pass_cols()(ya, yb, ei3)
    g, yn, cvec = _dense1_call(s_ab, xpad, dv16, dg16, w1p, b1[None], W2,
                               W3, W4, b2[None], b3[None])
    s2 = _pass_edge()(yn, ei3)
    z2, y2 = _comb_y()(s2, g, dv16, dg16, cvec[0])
    s3 = _pass_edge()(y2, ei3)
    z3, y3 = _comb_y()(s3, z2, dv16, dg16, cvec[1])
    s4 = _pass_edge()(y3, ei3)
    (h4,) = _comb_final()(s4, z3, dv16, dg16, b4)
    return h4[:N]
